# Initial kernel scaffold; baseline (speedup 1.0000x reference)
import functools

import jax
import jax.numpy as jnp
from jax import lax
from jax.experimental import pallas as pl
from jax.experimental.pallas import tpu as pltpu
from jax.experimental.pallas import tpu_sc as plsc

N = 10000
E = 320000
C = 128
NCORE = 2
NSUB = 16
NW = NCORE * NSUB
EPW = E // NW
B = 80
NCH = EPW // B
RPT = N // NSUB
AUXW = 16


_TC_R = 1000


def _mm_body(x_ref, w_ref, a_ref, h_ref, esed_ref):
    h = jnp.dot(x_ref[...], w_ref[...], preferred_element_type=jnp.float32)
    h_ref[...] = h
    esed_ref[...] = jnp.dot(h, a_ref[...], preferred_element_type=jnp.float32)


def _tc_in(x, W, A):
    return pl.pallas_call(
        _mm_body,
        grid=(N // _TC_R,),
        in_specs=[
            pl.BlockSpec((_TC_R, C), lambda i: (i, 0)),
            pl.BlockSpec((C, C), lambda i: (0, 0)),
            pl.BlockSpec((C, 2), lambda i: (0, 0)),
        ],
        out_specs=[
            pl.BlockSpec((_TC_R, C), lambda i: (i, 0)),
            pl.BlockSpec((_TC_R, 2), lambda i: (i, 0)),
        ],
        out_shape=[
            jax.ShapeDtypeStruct((N, C), jnp.float32),
            jax.ShapeDtypeStruct((N, 2), jnp.float32),
        ],
    )(x, W, A)


def _merge_rows(p128_ref, paux_ref, b_ref):
    s = p128_ref[0] + p128_ref[1]
    den = paux_ref[0, :, 0:1] + paux_ref[1, :, 0:1]
    x = s / (den + 1e-16) + b_ref[...]
    return jnp.maximum(x, 0.0)


def _merge_mm_body(p128_ref, paux_ref, b_ref, w_ref, a_ref, h_ref, esed_ref):
    x = _merge_rows(p128_ref, paux_ref, b_ref)
    h = jnp.dot(x, w_ref[...], preferred_element_type=jnp.float32)
    h_ref[...] = h
    esed_ref[...] = jnp.dot(h, a_ref[...], preferred_element_type=jnp.float32)


def _tc_merge_mm(p128, paux, b, W, A):
    return pl.pallas_call(
        _merge_mm_body,
        grid=(N // _TC_R,),
        in_specs=[
            pl.BlockSpec((2, _TC_R, C), lambda i: (0, i, 0)),
            pl.BlockSpec((2, _TC_R, AUXW), lambda i: (0, i, 0)),
            pl.BlockSpec((1, C), lambda i: (0, 0)),
            pl.BlockSpec((C, C), lambda i: (0, 0)),
            pl.BlockSpec((C, 2), lambda i: (0, 0)),
        ],
        out_specs=[
            pl.BlockSpec((_TC_R, C), lambda i: (i, 0)),
            pl.BlockSpec((_TC_R, 2), lambda i: (i, 0)),
        ],
        out_shape=[
            jax.ShapeDtypeStruct((N, C), jnp.float32),
            jax.ShapeDtypeStruct((N, 2), jnp.float32),
        ],
    )(p128, paux, b, W, A)


def _merge_fc_body(p128_ref, paux_ref, b_ref, wfc_ref, bfc_ref, out_ref):
    x = _merge_rows(p128_ref, paux_ref, b_ref)
    y = jnp.dot(x, wfc_ref[...], preferred_element_type=jnp.float32)
    out_ref[...] = jax.nn.sigmoid(y + bfc_ref[...])


def _tc_merge_fc(p128, paux, b, Wfc, bfc):
    return pl.pallas_call(
        _merge_fc_body,
        grid=(N // _TC_R,),
        in_specs=[
            pl.BlockSpec((2, _TC_R, C), lambda i: (0, i, 0)),
            pl.BlockSpec((2, _TC_R, AUXW), lambda i: (0, i, 0)),
            pl.BlockSpec((1, C), lambda i: (0, 0)),
            pl.BlockSpec((C, 1), lambda i: (0, 0)),
            pl.BlockSpec((1, 1), lambda i: (0, 0)),
        ],
        out_specs=pl.BlockSpec((_TC_R, 1), lambda i: (i, 0)),
        out_shape=jax.ShapeDtypeStruct((N, 1), jnp.float32),
    )(p128, paux, b, Wfc, bfc)


_SC_MESH = plsc.VectorSubcoreMesh(
    core_axis_name="c", subcore_axis_name="s", num_cores=NCORE, num_subcores=NSUB
)


@functools.partial(
    pl.kernel,
    mesh=_SC_MESH,
    out_type=[
        jax.ShapeDtypeStruct((NCORE, N, C), jnp.float32),
        jax.ShapeDtypeStruct((NCORE, N, AUXW), jnp.float32),
    ],
    scratch_types=[
        pltpu.VMEM((2 * N,), jnp.float32),
        pltpu.VMEM((NCH, B), jnp.int32),
        pltpu.VMEM((NCH, B), jnp.int32),
        pltpu.VMEM((NCH, B), jnp.float32),
        pltpu.VMEM((B, C), jnp.float32),
        pltpu.VMEM((B, AUXW), jnp.float32),
        pltpu.VMEM((B,), jnp.float32),
        pltpu.VMEM_SHARED((N, C), jnp.float32),
        pltpu.VMEM_SHARED((N, AUXW), jnp.float32),
        pltpu.SemaphoreType.DMA,
    ],
)
def _sc_edges(h_hbm, esed_hbm, src_hbm, dst_hbm, w_hbm, z128_hbm, zaux_hbm,
              p128_out, paux_out,
              esed_v, src_v, dst_v, w_v, rows_v, aux_v, p_v, acc_v, accaux_v,
              sem):
    cid = lax.axis_index("c")
    sid = lax.axis_index("s")
    wid = cid * NSUB + sid

    pltpu.sync_copy(esed_hbm, esed_v)
    pltpu.sync_copy(src_hbm.at[wid], src_v)
    pltpu.sync_copy(dst_hbm.at[wid], dst_v)
    pltpu.sync_copy(w_hbm.at[wid], w_v)

    rs = pl.ds(sid * RPT, RPT)
    pltpu.sync_copy(z128_hbm, acc_v.at[rs])
    pltpu.sync_copy(zaux_hbm, accaux_v.at[rs])

    def zero_aux(j, carry):
        aux_v[j, pl.ds(0, AUXW)] = jnp.zeros((AUXW,), jnp.float32)
        return carry

    lax.fori_loop(0, B, zero_aux, 0)
    plsc.subcore_barrier()

    lane = lax.iota(jnp.int32, 16)
    zero16 = jnp.zeros((16,), jnp.int32)

    def chunk(c, carry):
        gat = pltpu.async_copy(h_hbm.at[src_v.at[c]], rows_v, sem)
        for g in range(B // 16):
            sl = pl.ds(g * 16, 16)
            s_i = src_v[c, sl]
            d_i = dst_v[c, sl]
            es = plsc.load_gather(esed_v, [s_i * 2])
            ed = plsc.load_gather(esed_v, [d_i * 2 + 1])
            e = es + ed
            e = jnp.where(e >= 0.0, e, 0.2 * e)
            ex = jnp.exp(e)
            p_v[sl] = ex * w_v[c, sl]
            plsc.store_scatter(aux_v, [g * 16 + lane, zero16], ex)
        gat.wait()

        def scale(j, carry2):
            pj = p_v[j]
            for k in range(C // 16):
                ksl = pl.ds(k * 16, 16)
                rows_v[j, ksl] = rows_v[j, ksl] * pj
            return carry2

        lax.fori_loop(0, B, scale, 0)

        pltpu.sync_copy(rows_v, acc_v.at[dst_v.at[c]], add=True)
        pltpu.sync_copy(aux_v, accaux_v.at[dst_v.at[c]], add=True)
        return carry

    lax.fori_loop(0, NCH, chunk, 0)
    plsc.subcore_barrier()

    pltpu.sync_copy(acc_v.at[rs], p128_out.at[cid, rs])
    pltpu.sync_copy(accaux_v.at[rs], paux_out.at[cid, rs])


def kernel(_x, _edge_index, _edge_weight, W1, a1_src, a1_dst, b1,
           W2, a2_src, a2_dst, b2, Wfc, bfc):
    src = _edge_index[0].reshape(NW, NCH, B)
    dst = _edge_index[1].reshape(NW, NCH, B)
    w = _edge_weight.reshape(NW, NCH, B)
    A1 = jnp.concatenate([a1_src.reshape(C, 1), a1_dst.reshape(C, 1)], axis=1)
    A2 = jnp.concatenate([a2_src.reshape(C, 1), a2_dst.reshape(C, 1)], axis=1)
    z128 = jnp.zeros((RPT, C), jnp.float32)
    zaux = jnp.zeros((RPT, AUXW), jnp.float32)

    h1, esed1 = _tc_in(_x, W1, A1)
    p128_1, paux_1 = _sc_edges(h1, esed1.reshape(2 * N), src, dst, w, z128, zaux)
    h2, esed2 = _tc_merge_mm(p128_1, paux_1, b1.reshape(1, C), W2, A2)
    p128_2, paux_2 = _sc_edges(h2, esed2.reshape(2 * N), src, dst, w, z128, zaux)
    return _tc_merge_fc(p128_2, paux_2, b2.reshape(1, C), Wfc.reshape(C, 1),
                        bfc.reshape(1, 1))


# speedup vs baseline: 27.5891x; 27.5891x over previous
import functools

import jax
import jax.numpy as jnp
from jax import lax
from jax.experimental import pallas as pl
from jax.experimental.pallas import tpu as pltpu
from jax.experimental.pallas import tpu_sc as plsc

N = 10000
E = 320000
C = 128
NCORE = 2
NSUB = 16
NW = NCORE * NSUB
EPW = E // NW
B = 80
NCH = EPW // B
NP = 10240
RPT = NP // NSUB
AUXW = 16


_TC_R = 1000


def _mm_body(x_ref, w_ref, a_ref, h_ref, esed_ref):
    h = jnp.dot(x_ref[...], w_ref[...], preferred_element_type=jnp.float32)
    h_ref[...] = h
    esed_ref[...] = jnp.dot(h, a_ref[...], preferred_element_type=jnp.float32)


def _tc_in(x, W, A):
    return pl.pallas_call(
        _mm_body,
        grid=(N // _TC_R,),
        in_specs=[
            pl.BlockSpec((_TC_R, C), lambda i: (i, 0)),
            pl.BlockSpec((C, C), lambda i: (0, 0)),
            pl.BlockSpec((C, 2), lambda i: (0, 0)),
        ],
        out_specs=[
            pl.BlockSpec((_TC_R, C), lambda i: (i, 0)),
            pl.BlockSpec((_TC_R, 2), lambda i: (i, 0)),
        ],
        out_shape=[
            jax.ShapeDtypeStruct((N, C), jnp.float32),
            jax.ShapeDtypeStruct((N, 2), jnp.float32),
        ],
    )(x, W, A)


def _merge_rows(p128_ref, paux_ref, b_ref):
    s = p128_ref[0] + p128_ref[1]
    den = paux_ref[0, :, 0:1] + paux_ref[1, :, 0:1]
    x = s / (den + 1e-16) + b_ref[...]
    return jnp.maximum(x, 0.0)


def _merge_mm_body(p128_ref, paux_ref, b_ref, w_ref, a_ref, h_ref, esed_ref):
    x = _merge_rows(p128_ref, paux_ref, b_ref)
    h = jnp.dot(x, w_ref[...], preferred_element_type=jnp.float32)
    h_ref[...] = h
    esed_ref[...] = jnp.dot(h, a_ref[...], preferred_element_type=jnp.float32)


def _tc_merge_mm(p128, paux, b, W, A):
    return pl.pallas_call(
        _merge_mm_body,
        grid=(N // _TC_R,),
        in_specs=[
            pl.BlockSpec((2, _TC_R, C), lambda i: (0, i, 0)),
            pl.BlockSpec((2, _TC_R, AUXW), lambda i: (0, i, 0)),
            pl.BlockSpec((1, C), lambda i: (0, 0)),
            pl.BlockSpec((C, C), lambda i: (0, 0)),
            pl.BlockSpec((C, 2), lambda i: (0, 0)),
        ],
        out_specs=[
            pl.BlockSpec((_TC_R, C), lambda i: (i, 0)),
            pl.BlockSpec((_TC_R, 2), lambda i: (i, 0)),
        ],
        out_shape=[
            jax.ShapeDtypeStruct((N, C), jnp.float32),
            jax.ShapeDtypeStruct((N, 2), jnp.float32),
        ],
    )(p128, paux, b, W, A)


def _merge_fc_body(p128_ref, paux_ref, b_ref, wfc_ref, bfc_ref, out_ref):
    x = _merge_rows(p128_ref, paux_ref, b_ref)
    y = jnp.dot(x, wfc_ref[...], preferred_element_type=jnp.float32)
    out_ref[...] = jax.nn.sigmoid(y + bfc_ref[...])


def _tc_merge_fc(p128, paux, b, Wfc, bfc):
    return pl.pallas_call(
        _merge_fc_body,
        grid=(N // _TC_R,),
        in_specs=[
            pl.BlockSpec((2, _TC_R, C), lambda i: (0, i, 0)),
            pl.BlockSpec((2, _TC_R, AUXW), lambda i: (0, i, 0)),
            pl.BlockSpec((1, C), lambda i: (0, 0)),
            pl.BlockSpec((C, 1), lambda i: (0, 0)),
            pl.BlockSpec((1, 1), lambda i: (0, 0)),
        ],
        out_specs=pl.BlockSpec((_TC_R, 1), lambda i: (i, 0)),
        out_shape=jax.ShapeDtypeStruct((N, 1), jnp.float32),
    )(p128, paux, b, Wfc, bfc)


SB = 5
SCH = NCH // SB

_SC_MESH = plsc.VectorSubcoreMesh(
    core_axis_name="c", subcore_axis_name="s", num_cores=NCORE, num_subcores=NSUB
)

_SC_PARAMS = pltpu.CompilerParams(needs_layout_passes=False)


@functools.partial(
    pl.kernel,
    mesh=_SC_MESH,
    compiler_params=_SC_PARAMS,
    out_type=[
        jax.ShapeDtypeStruct((NW * SCH, SB, B), jnp.float32),
        jax.ShapeDtypeStruct((NCORE, NP, AUXW), jnp.float32),
    ],
    scratch_types=[
        pltpu.VMEM((2 * N,), jnp.float32),
        pltpu.VMEM((SB, B), jnp.int32),
        pltpu.VMEM((SB, B), jnp.int32),
        pltpu.VMEM((SB, B), jnp.float32),
        pltpu.VMEM((SB, B), jnp.float32),
        pltpu.VMEM((B, AUXW), jnp.float32),
        pltpu.VMEM_SHARED((NP, AUXW), jnp.float32),
    ],
)
def _sc_logits(esed_hbm, src_hbm, dst_hbm, w_hbm, zaux_hbm,
               p_out, paux_out,
               esed_v, src_v, dst_v, w_v, p_v, aux_v, accaux_v):
    cid = lax.axis_index("c")
    sid = lax.axis_index("s")
    wid = cid * NSUB + sid

    pltpu.sync_copy(esed_hbm, esed_v)
    rs = pl.ds(sid * RPT, RPT)
    pltpu.sync_copy(zaux_hbm, accaux_v.at[rs])

    def zero_aux(j, carry):
        aux_v[j, pl.ds(0, AUXW)] = jnp.zeros((AUXW,), jnp.float32)
        return carry

    lax.fori_loop(0, B, zero_aux, 0)
    plsc.subcore_barrier()

    lane = lax.iota(jnp.int32, 16)
    zero16 = jnp.zeros((16,), jnp.int32)

    def superchunk(sc, carry):
        wsc = wid * SCH + sc
        pltpu.sync_copy(src_hbm.at[wsc], src_v)
        pltpu.sync_copy(dst_hbm.at[wsc], dst_v)
        pltpu.sync_copy(w_hbm.at[wsc], w_v)
        for b in range(SB):
            for g in range(B // 16):
                sl = pl.ds(g * 16, 16)
                s_i = src_v[b, sl]
                d_i = dst_v[b, sl]
                es = plsc.load_gather(esed_v, [s_i * 2])
                ed = plsc.load_gather(esed_v, [d_i * 2 + 1])
                e = es + ed
                e = jnp.where(e >= 0.0, e, 0.2 * e)
                ex = jnp.exp(e)
                p_v[b, sl] = ex * w_v[b, sl]
                plsc.store_scatter(aux_v, [g * 16 + lane, zero16], ex)
            pltpu.sync_copy(aux_v, accaux_v.at[dst_v.at[b]], add=True)
        pltpu.sync_copy(p_v, p_out.at[wsc])
        return carry

    lax.fori_loop(0, SCH, superchunk, 0)
    plsc.subcore_barrier()
    pltpu.sync_copy(accaux_v.at[rs], paux_out.at[cid, rs])


@functools.partial(
    pl.kernel,
    mesh=_SC_MESH,
    compiler_params=_SC_PARAMS,
    out_type=jax.ShapeDtypeStruct((NCORE, NP, C), jnp.float32),
    scratch_types=[
        pltpu.VMEM((SB, B), jnp.int32),
        pltpu.VMEM((SB, B), jnp.int32),
        pltpu.VMEM((SB, B), jnp.float32),
        pltpu.VMEM((B, C), jnp.float32),
        pltpu.VMEM((B, C), jnp.float32),
        pltpu.VMEM_SHARED((NP, C), jnp.float32),
        pltpu.SemaphoreType.DMA,
        pltpu.SemaphoreType.DMA,
    ],
)
def _sc_messages(h_hbm, src_hbm, dst_hbm, p_hbm, z128_hbm,
                 p128_out,
                 src_v, dst_v, p_v, rows0_v, rows1_v, acc_v, sem0, sem1):
    cid = lax.axis_index("c")
    sid = lax.axis_index("s")
    wid = cid * NSUB + sid

    rs = pl.ds(sid * RPT, RPT)
    pltpu.sync_copy(z128_hbm, acc_v.at[rs])
    plsc.subcore_barrier()

    rows = (rows0_v, rows1_v)
    sems = (sem0, sem1)

    def superchunk(sc, carry):
        wsc = wid * SCH + sc
        pltpu.sync_copy(src_hbm.at[wsc], src_v)
        pltpu.sync_copy(dst_hbm.at[wsc], dst_v)
        pltpu.sync_copy(p_hbm.at[wsc], p_v)
        gat = pltpu.async_copy(h_hbm.at[src_v.at[0]], rows[0], sems[0])
        for b in range(SB):
            if b + 1 < SB:
                nxt = pltpu.async_copy(
                    h_hbm.at[src_v.at[b + 1]], rows[(b + 1) % 2],
                    sems[(b + 1) % 2])
            gat.wait()
            cur = rows[b % 2]

            def scale(g, carry2, _b=b, _cur=cur):
                pv = p_v[_b, pl.ds(g * 16, 16)]
                for l in range(16):
                    pj = pv[l]
                    for k in range(C // 16):
                        ksl = pl.ds(k * 16, 16)
                        _cur[g * 16 + l, ksl] = _cur[g * 16 + l, ksl] * pj
                return carry2

            lax.fori_loop(0, B // 16, scale, 0)
            pltpu.sync_copy(cur, acc_v.at[dst_v.at[b]], add=True)
            if b + 1 < SB:
                gat = nxt
        return carry

    lax.fori_loop(0, SCH, superchunk, 0)
    plsc.subcore_barrier()
    pltpu.sync_copy(acc_v.at[rs], p128_out.at[cid, rs])


def _gat_edge_phase(h, esed, src, dst, w, z128, zaux):
    p, paux = _sc_logits(esed.reshape(2 * N), src, dst, w, zaux)
    p128 = _sc_messages(h, src, dst, p, z128)
    return p128, paux


def kernel(_x, _edge_index, _edge_weight, W1, a1_src, a1_dst, b1,
           W2, a2_src, a2_dst, b2, Wfc, bfc):
    src = _edge_index[0].reshape(NW * SCH, SB, B)
    dst = _edge_index[1].reshape(NW * SCH, SB, B)
    w = _edge_weight.reshape(NW * SCH, SB, B)
    A1 = jnp.concatenate([a1_src.reshape(C, 1), a1_dst.reshape(C, 1)], axis=1)
    A2 = jnp.concatenate([a2_src.reshape(C, 1), a2_dst.reshape(C, 1)], axis=1)
    z128 = jnp.zeros((RPT, C), jnp.float32)
    zaux = jnp.zeros((RPT, AUXW), jnp.float32)

    h1, esed1 = _tc_in(_x, W1, A1)
    p128_1, paux_1 = _gat_edge_phase(h1, esed1, src, dst, w, z128, zaux)
    h2, esed2 = _tc_merge_mm(p128_1, paux_1, b1.reshape(1, C), W2, A2)
    p128_2, paux_2 = _gat_edge_phase(h2, esed2, src, dst, w, z128, zaux)
    return _tc_merge_fc(p128_2, paux_2, b2.reshape(1, C), Wfc.reshape(C, 1),
                        bfc.reshape(1, 1))


# speedup vs baseline: 34.2158x; 1.2402x over previous
import functools

import jax
import jax.numpy as jnp
from jax import lax
from jax.experimental import pallas as pl
from jax.experimental.pallas import tpu as pltpu
from jax.experimental.pallas import tpu_sc as plsc

N = 10000
E = 320000
C = 128
NCORE = 2
NSUB = 16
NW = NCORE * NSUB
EPW = E // NW
B = 80
NCH = EPW // B
NP = 10240
RPT = NP // NSUB
AUXW = 16


_TC_R = 1000


def _mm_body(x_ref, w_ref, a_ref, h_ref, esed_ref):
    h = jnp.dot(x_ref[...], w_ref[...], preferred_element_type=jnp.float32)
    h_ref[...] = h
    esed_ref[...] = jnp.dot(h, a_ref[...], preferred_element_type=jnp.float32)


def _tc_in(x, W, A):
    return pl.pallas_call(
        _mm_body,
        grid=(N // _TC_R,),
        in_specs=[
            pl.BlockSpec((_TC_R, C), lambda i: (i, 0)),
            pl.BlockSpec((C, C), lambda i: (0, 0)),
            pl.BlockSpec((C, 2), lambda i: (0, 0)),
        ],
        out_specs=[
            pl.BlockSpec((_TC_R, C), lambda i: (i, 0)),
            pl.BlockSpec((_TC_R, 2), lambda i: (i, 0)),
        ],
        out_shape=[
            jax.ShapeDtypeStruct((N, C), jnp.float32),
            jax.ShapeDtypeStruct((N, 2), jnp.float32),
        ],
    )(x, W, A)


def _merge_rows(p128_ref, paux_ref, b_ref):
    s = p128_ref[0] + p128_ref[1]
    den = paux_ref[0, :, 0:1] + paux_ref[1, :, 0:1]
    x = s / (den + 1e-16) + b_ref[...]
    return jnp.maximum(x, 0.0)


def _merge_mm_body(p128_ref, paux_ref, b_ref, w_ref, a_ref, h_ref, esed_ref):
    x = _merge_rows(p128_ref, paux_ref, b_ref)
    h = jnp.dot(x, w_ref[...], preferred_element_type=jnp.float32)
    h_ref[...] = h
    esed_ref[...] = jnp.dot(h, a_ref[...], preferred_element_type=jnp.float32)


def _tc_merge_mm(p128, paux, b, W, A):
    return pl.pallas_call(
        _merge_mm_body,
        grid=(N // _TC_R,),
        in_specs=[
            pl.BlockSpec((2, _TC_R, C), lambda i: (0, i, 0)),
            pl.BlockSpec((2, _TC_R, AUXW), lambda i: (0, i, 0)),
            pl.BlockSpec((1, C), lambda i: (0, 0)),
            pl.BlockSpec((C, C), lambda i: (0, 0)),
            pl.BlockSpec((C, 2), lambda i: (0, 0)),
        ],
        out_specs=[
            pl.BlockSpec((_TC_R, C), lambda i: (i, 0)),
            pl.BlockSpec((_TC_R, 2), lambda i: (i, 0)),
        ],
        out_shape=[
            jax.ShapeDtypeStruct((N, C), jnp.float32),
            jax.ShapeDtypeStruct((N, 2), jnp.float32),
        ],
    )(p128, paux, b, W, A)


def _merge_fc_body(p128_ref, paux_ref, b_ref, wfc_ref, bfc_ref, out_ref):
    x = _merge_rows(p128_ref, paux_ref, b_ref)
    y = jnp.dot(x, wfc_ref[...], preferred_element_type=jnp.float32)
    out_ref[...] = jax.nn.sigmoid(y + bfc_ref[...])


def _tc_merge_fc(p128, paux, b, Wfc, bfc):
    return pl.pallas_call(
        _merge_fc_body,
        grid=(N // _TC_R,),
        in_specs=[
            pl.BlockSpec((2, _TC_R, C), lambda i: (0, i, 0)),
            pl.BlockSpec((2, _TC_R, AUXW), lambda i: (0, i, 0)),
            pl.BlockSpec((1, C), lambda i: (0, 0)),
            pl.BlockSpec((C, 1), lambda i: (0, 0)),
            pl.BlockSpec((1, 1), lambda i: (0, 0)),
        ],
        out_specs=pl.BlockSpec((_TC_R, 1), lambda i: (i, 0)),
        out_shape=jax.ShapeDtypeStruct((N, 1), jnp.float32),
    )(p128, paux, b, Wfc, bfc)


SB = 25
SCH = NCH // SB

_SC_MESH = plsc.VectorSubcoreMesh(
    core_axis_name="c", subcore_axis_name="s", num_cores=NCORE, num_subcores=NSUB
)

_SC_PARAMS = pltpu.CompilerParams(needs_layout_passes=False)


@functools.partial(
    pl.kernel,
    mesh=_SC_MESH,
    compiler_params=_SC_PARAMS,
    out_type=[
        jax.ShapeDtypeStruct((NW * SCH, SB, B), jnp.float32),
        jax.ShapeDtypeStruct((NCORE, NP, AUXW), jnp.float32),
    ],
    scratch_types=[
        pltpu.VMEM((2 * N,), jnp.float32),
        pltpu.VMEM((SB, B), jnp.int32),
        pltpu.VMEM((SB, B), jnp.int32),
        pltpu.VMEM((SB, B), jnp.float32),
        pltpu.VMEM((SB, B), jnp.float32),
        pltpu.VMEM((B, AUXW), jnp.float32),
        pltpu.VMEM_SHARED((NP, AUXW), jnp.float32),
    ],
)
def _sc_logits(esed_hbm, src_hbm, dst_hbm, w_hbm, zaux_hbm,
               p_out, paux_out,
               esed_v, src_v, dst_v, w_v, p_v, aux_v, accaux_v):
    cid = lax.axis_index("c")
    sid = lax.axis_index("s")
    wid = cid * NSUB + sid

    pltpu.sync_copy(esed_hbm, esed_v)
    rs = pl.ds(sid * RPT, RPT)
    pltpu.sync_copy(zaux_hbm, accaux_v.at[rs])

    def zero_aux(j, carry):
        aux_v[j, pl.ds(0, AUXW)] = jnp.zeros((AUXW,), jnp.float32)
        return carry

    lax.fori_loop(0, B, zero_aux, 0)
    plsc.subcore_barrier()

    lane = lax.iota(jnp.int32, 16)
    zero16 = jnp.zeros((16,), jnp.int32)

    def superchunk(sc, carry):
        wsc = wid * SCH + sc
        pltpu.sync_copy(src_hbm.at[wsc], src_v)
        pltpu.sync_copy(dst_hbm.at[wsc], dst_v)
        pltpu.sync_copy(w_hbm.at[wsc], w_v)
        for b in range(SB):
            for g in range(B // 16):
                sl = pl.ds(g * 16, 16)
                s_i = src_v[b, sl]
                d_i = dst_v[b, sl]
                es = plsc.load_gather(esed_v, [s_i * 2])
                ed = plsc.load_gather(esed_v, [d_i * 2 + 1])
                e = es + ed
                e = jnp.where(e >= 0.0, e, 0.2 * e)
                ex = jnp.exp(e)
                p_v[b, sl] = ex * w_v[b, sl]
                plsc.store_scatter(aux_v, [g * 16 + lane, zero16], ex)
            pltpu.sync_copy(aux_v, accaux_v.at[dst_v.at[b]], add=True)
        pltpu.sync_copy(p_v, p_out.at[wsc])
        return carry

    lax.fori_loop(0, SCH, superchunk, 0)
    plsc.subcore_barrier()
    pltpu.sync_copy(accaux_v.at[rs], paux_out.at[cid, rs])


@functools.partial(
    pl.kernel,
    mesh=_SC_MESH,
    compiler_params=_SC_PARAMS,
    out_type=jax.ShapeDtypeStruct((NCORE, NP, C), jnp.float32),
    scratch_types=[
        pltpu.VMEM((SB, B), jnp.int32),
        pltpu.VMEM((SB, B), jnp.int32),
        pltpu.VMEM((SB, B), jnp.float32),
        pltpu.VMEM((B, C), jnp.float32),
        pltpu.VMEM((B, C), jnp.float32),
        pltpu.VMEM_SHARED((NP, C), jnp.float32),
        pltpu.SemaphoreType.DMA,
        pltpu.SemaphoreType.DMA,
    ],
)
def _sc_messages(h_hbm, src_hbm, dst_hbm, p_hbm, z128_hbm,
                 p128_out,
                 src_v, dst_v, p_v, rows0_v, rows1_v, acc_v, sem0, sem1):
    cid = lax.axis_index("c")
    sid = lax.axis_index("s")
    wid = cid * NSUB + sid

    rs = pl.ds(sid * RPT, RPT)
    pltpu.sync_copy(z128_hbm, acc_v.at[rs])
    plsc.subcore_barrier()

    rows = (rows0_v, rows1_v)
    sems = (sem0, sem1)

    def superchunk(sc, carry):
        wsc = wid * SCH + sc
        pltpu.sync_copy(src_hbm.at[wsc], src_v)
        pltpu.sync_copy(dst_hbm.at[wsc], dst_v)
        pltpu.sync_copy(p_hbm.at[wsc], p_v)
        gat = pltpu.async_copy(h_hbm.at[src_v.at[0]], rows[0], sems[0])
        for b in range(SB):
            if b + 1 < SB:
                nxt = pltpu.async_copy(
                    h_hbm.at[src_v.at[b + 1]], rows[(b + 1) % 2],
                    sems[(b + 1) % 2])
            gat.wait()
            cur = rows[b % 2]

            def scale(g, carry2, _b=b, _cur=cur):
                pv = p_v[_b, pl.ds(g * 16, 16)]
                for l in range(16):
                    pj = pv[l]
                    for k in range(C // 16):
                        ksl = pl.ds(k * 16, 16)
                        _cur[g * 16 + l, ksl] = _cur[g * 16 + l, ksl] * pj
                return carry2

            lax.fori_loop(0, B // 16, scale, 0)
            pltpu.sync_copy(cur, acc_v.at[dst_v.at[b]], add=True)
            if b + 1 < SB:
                gat = nxt
        return carry

    lax.fori_loop(0, SCH, superchunk, 0)
    plsc.subcore_barrier()
    pltpu.sync_copy(acc_v.at[rs], p128_out.at[cid, rs])


def _gat_edge_phase(h, esed, src, dst, w, z128, zaux):
    p, paux = _sc_logits(esed.reshape(2 * N), src, dst, w, zaux)
    p128 = _sc_messages(h, src, dst, p, z128)
    return p128, paux


def kernel(_x, _edge_index, _edge_weight, W1, a1_src, a1_dst, b1,
           W2, a2_src, a2_dst, b2, Wfc, bfc):
    src = _edge_index[0].reshape(NW * SCH, SB, B)
    dst = _edge_index[1].reshape(NW * SCH, SB, B)
    w = _edge_weight.reshape(NW * SCH, SB, B)
    A1 = jnp.concatenate([a1_src.reshape(C, 1), a1_dst.reshape(C, 1)], axis=1)
    A2 = jnp.concatenate([a2_src.reshape(C, 1), a2_dst.reshape(C, 1)], axis=1)
    z128 = jnp.zeros((RPT, C), jnp.float32)
    zaux = jnp.zeros((RPT, AUXW), jnp.float32)

    h1, esed1 = _tc_in(_x, W1, A1)
    p128_1, paux_1 = _gat_edge_phase(h1, esed1, src, dst, w, z128, zaux)
    h2, esed2 = _tc_merge_mm(p128_1, paux_1, b1.reshape(1, C), W2, A2)
    p128_2, paux_2 = _gat_edge_phase(h2, esed2, src, dst, w, z128, zaux)
    return _tc_merge_fc(p128_2, paux_2, b2.reshape(1, C), Wfc.reshape(C, 1),
                        bfc.reshape(1, 1))
